# baseline (speedup 1.0000x reference)
import jax
import jax.numpy as jnp
from jax.experimental import pallas as pl
from jax.experimental.pallas import tpu as pltpu


_CHUNK = 10000
_NBUF = 4


def _node_table_kernel(x_hbm, emb_hbm, w_ref, b_ref, out_hbm,
                       x_buf, emb_buf, out_buf, x_sem, emb_sem, out_sem):
    n = x_hbm.shape[0]
    nchunks = n // _CHUNK

    def in_copies(i, slot):
        rows = pl.ds(i * _CHUNK, _CHUNK)
        return (
            pltpu.make_async_copy(x_hbm.at[rows], x_buf.at[slot], x_sem.at[slot]),
            pltpu.make_async_copy(emb_hbm.at[rows], emb_buf.at[slot], emb_sem.at[slot]),
        )

    def out_copy(i, slot):
        rows = pl.ds(i * _CHUNK, _CHUNK)
        return pltpu.make_async_copy(out_buf.at[slot], out_hbm.at[rows], out_sem.at[slot])

    for i in range(_NBUF):
        for c in in_copies(i, i):
            c.start()

    w = w_ref[...]
    b = b_ref[...]

    for i in range(nchunks):
        slot = i % _NBUF
        for c in in_copies(i, slot):
            c.wait()
        if i >= _NBUF:
            out_copy(i - _NBUF, slot).wait()
        x = x_buf[slot]
        proj = jax.lax.dot_general(
            x, w,
            dimension_numbers=(((1,), (1,)), ((), ())),
            preferred_element_type=jnp.float32,
        )
        out_buf[slot] = proj + b + emb_buf[slot]
        out_copy(i, slot).start()
        if i + _NBUF < nchunks:
            for c in in_copies(i + _NBUF, slot):
                c.start()

    for i in range(nchunks - _NBUF, nchunks):
        out_copy(i, i % _NBUF).wait()


def kernel(node_features, emb_weight, proj_w, proj_b):
    n, d = node_features.shape
    b2d = proj_b.reshape(1, -1)
    return pl.pallas_call(
        _node_table_kernel,
        in_specs=[
            pl.BlockSpec(memory_space=pl.ANY),
            pl.BlockSpec(memory_space=pl.ANY),
            pl.BlockSpec(memory_space=pltpu.MemorySpace.VMEM),
            pl.BlockSpec(memory_space=pltpu.MemorySpace.VMEM),
        ],
        out_specs=pl.BlockSpec(memory_space=pl.ANY),
        out_shape=jax.ShapeDtypeStruct((n, d), jnp.float32),
        scratch_shapes=[
            pltpu.VMEM((_NBUF, _CHUNK, d), jnp.float32),
            pltpu.VMEM((_NBUF, _CHUNK, d), jnp.float32),
            pltpu.VMEM((_NBUF, _CHUNK, d), jnp.float32),
            pltpu.SemaphoreType.DMA((_NBUF,)),
            pltpu.SemaphoreType.DMA((_NBUF,)),
            pltpu.SemaphoreType.DMA((_NBUF,)),
        ],
        compiler_params=pltpu.CompilerParams(
            vmem_limit_bytes=67_000_000,
        ),
    )(node_features, emb_weight, proj_w, b2d)


# speedup vs baseline: 1.0270x; 1.0270x over previous
import jax
import jax.numpy as jnp
from jax.experimental import pallas as pl
from jax.experimental.pallas import tpu as pltpu


_BLOCK_ROWS = 10000
_INNER = 5


def _node_table_kernel(x_ref, emb_ref, w_ref, b_ref, out_ref):
    x = x_ref[...]
    proj = jax.lax.dot_general(
        x, w_ref[...],
        dimension_numbers=(((1,), (1,)), ((), ())),
        preferred_element_type=jnp.float32,
    )
    out_ref[...] = proj + b_ref[...] + emb_ref[...]


def kernel(node_features, emb_weight, proj_w, proj_b):
    n, d = node_features.shape
    nblocks = -(-n // _BLOCK_ROWS)
    outer = nblocks // _INNER
    b2d = proj_b.reshape(1, -1)
    blk = lambda c, i: (c * _INNER + i, 0)
    rep = lambda c, i: (0, 0)
    return pl.pallas_call(
        _node_table_kernel,
        grid=(outer, _INNER),
        in_specs=[
            pl.BlockSpec((_BLOCK_ROWS, d), blk),
            pl.BlockSpec((_BLOCK_ROWS, d), blk),
            pl.BlockSpec(proj_w.shape, rep),
            pl.BlockSpec((1, d), rep),
        ],
        out_specs=pl.BlockSpec((_BLOCK_ROWS, d), blk),
        out_shape=jax.ShapeDtypeStruct((n, d), jnp.float32),
        compiler_params=pltpu.CompilerParams(
            dimension_semantics=("parallel", "arbitrary"),
            vmem_limit_bytes=67_000_000,
        ),
    )(node_features, emb_weight, proj_w, b2d)
